# baseline (speedup 1.0000x reference)
import functools

import jax
import jax.numpy as jnp
from jax import lax
from jax.experimental import pallas as pl
from jax.experimental.pallas import tpu as pltpu
from jax.experimental.pallas import tpu_sc as plsc

T, D, FF, E, K = 4096, 1024, 4096, 8, 2
PAIRS = T * K
RB = 256
NRB = T // RB
TM2 = 128
R = PAIRS + E * TM2
NT = R // TM2
FC = 1024
NFC = FF // FC
NW = 32
CW = R // NW
CP = PAIRS // NW
TMA = 512


def _router_meta_body(x_ref, wg_ref, logits_ref, pos_ref, wp_ref, gid_ref,
                      xb_ref, cnt_ref, nxt_ref, idx1_ref, idx2_ref, wa_ref):
    g = pl.program_id(0)
    eio = lax.broadcasted_iota(jnp.int32, (RB, E), 1)

    @pl.when(g == 0)
    def _():
        cnt_ref[...] = jnp.zeros_like(cnt_ref)

    @pl.when(g < NRB)
    def _():
        x = x_ref[...]
        xb_ref[...] = x.astype(jnp.bfloat16)
        logits = jnp.dot(x, wg_ref[...], preferred_element_type=jnp.float32)
        logits_ref[...] = logits
        m1 = jnp.max(logits, axis=1, keepdims=True)
        idx1 = jnp.min(jnp.where(logits == m1, eio, E), axis=1, keepdims=True)
        masked = jnp.where(eio == idx1, -1e30, logits)
        m2 = jnp.max(masked, axis=1, keepdims=True)
        idx2 = jnp.min(jnp.where(masked == m2, eio, E), axis=1, keepdims=True)
        wa = 1.0 / (1.0 + jnp.exp(m2 - m1))
        sl = pl.ds(g * RB, RB)
        idx1_ref[sl] = idx1
        idx2_ref[sl] = idx2
        wa_ref[sl] = wa
        oh = (eio == idx1).astype(jnp.float32) + (eio == idx2).astype(jnp.float32)
        cnt_ref[...] += jnp.sum(oh, axis=0, keepdims=True)
        pos_ref[...] = jnp.zeros((RB, 1), jnp.int32)
        wp_ref[...] = jnp.zeros((RB, 1), jnp.float32)

    @pl.when(g == NRB)
    def _():
        cnt = cnt_ref[...]
        pads = jnp.floor((cnt + (TM2 - 1)) * (1.0 / TM2)) * TM2
        rio8 = lax.broadcasted_iota(jnp.int32, (E, E), 0)
        cio8 = lax.broadcasted_iota(jnp.int32, (E, E), 1)
        mlt = (rio8 < cio8).astype(jnp.float32)
        excl = jnp.dot(pads, mlt, preferred_element_type=jnp.float32)
        nxt_ref[...] = excl
        tio = lax.broadcasted_iota(jnp.int32, (1, 128), 1).astype(jnp.float32) * TM2
        gid = jnp.zeros((1, 128), jnp.float32)
        for e in range(1, E):
            gid += (tio >= excl[0:1, e:e + 1]).astype(jnp.float32)
        gid_ref[...] = gid.astype(jnp.int32)

    @pl.when(g >= NRB)
    def _():
        j = g - NRB
        slot = j // NRB
        sl = pl.ds(lax.rem(j, NRB) * RB, RB)
        idx1 = idx1_ref[sl]
        idx2 = idx2_ref[sl]
        wa = wa_ref[sl]
        f = jnp.where(slot == 0, idx1, idx2)
        wsel = jnp.where(slot == 0, wa, 1.0 - wa)
        oh = (eio == f).astype(jnp.float32)
        rio = lax.broadcasted_iota(jnp.int32, (RB, RB), 0)
        cio = lax.broadcasted_iota(jnp.int32, (RB, RB), 1)
        lstrict = (cio < rio).astype(jnp.float32)
        csum = jnp.dot(lstrict, oh, preferred_element_type=jnp.float32)
        rank = jnp.sum(csum * oh, axis=1, keepdims=True)
        base = jnp.sum(nxt_ref[...] * oh, axis=1, keepdims=True)
        pos_ref[...] = (rank + base).astype(jnp.int32)
        wp_ref[...] = wsel
        nxt_ref[...] += jnp.sum(oh, axis=0, keepdims=True)


def _scatter_body(pos_hbm, wp_hbm, tok_hbm, zi_hbm, zf_hbm,
                  tokrow_hbm, wrow_hbm,
                  pos_v, wp_v, tok_v, dtok_v, dw_v):
    c = lax.axis_index("c")
    s = lax.axis_index("s")
    wid = s * 2 + c

    @pl.when(wid == 0)
    def _():
        pltpu.sync_copy(pos_hbm, pos_v)
        pltpu.sync_copy(wp_hbm, wp_v)
        pltpu.sync_copy(tok_hbm, tok_v)
        pltpu.sync_copy(zi_hbm, dtok_v)
        pltpu.sync_copy(zf_hbm, dw_v)

        def sbody(i, carry):
            sl = pl.ds(i * 16, 16)
            idx = pos_v[sl]
            plsc.store_scatter(dtok_v, [idx], tok_v[sl])
            plsc.store_scatter(dw_v, [idx], wp_v[sl])
            return carry

        lax.fori_loop(0, PAIRS // 16, sbody, 0)
        pltpu.sync_copy(dtok_v, tokrow_hbm)
        pltpu.sync_copy(dw_v, wrow_hbm)


NBUF = 3


def _gather_body(rows_w, x_hbm, tok_hbm, xs_hbm, idx_v, buf_v,
                 semg0, semg1, semg2, semo0, semo1, semo2):
    c = lax.axis_index("c")
    s = lax.axis_index("s")
    wid = s * 2 + c
    base = wid * rows_w
    pltpu.sync_copy(tok_hbm.at[pl.ds(base, rows_w)], idx_v)
    semg = (semg0, semg1, semg2)
    semo = (semo0, semo1, semo2)
    nch = rows_w // 32
    gds = [None] * NBUF
    ods = [None] * NBUF
    for ch in range(nch):
        b = ch % NBUF
        if ods[b] is not None:
            ods[b].wait()
        gds[b] = pltpu.async_copy(
            x_hbm.at[idx_v.at[pl.ds(ch * 32, 32)]], buf_v.at[b], semg[b])
        old = ch - (NBUF - 1)
        if old >= 0:
            ob = old % NBUF
            gds[ob].wait()
            ods[ob] = pltpu.async_copy(
                buf_v.at[ob], xs_hbm.at[pl.ds(base + old * 32, 32)], semo[ob])
    for old in range(max(0, nch - (NBUF - 1)), nch):
        ob = old % NBUF
        gds[ob].wait()
        ods[ob] = pltpu.async_copy(
            buf_v.at[ob], xs_hbm.at[pl.ds(base + old * 32, 32)], semo[ob])
    for b in range(NBUF):
        if ods[b] is not None:
            ods[b].wait()


def _gmm_body(gid_ref, xs_ref, w1_ref, w3_ref, w2_ref, wrow_ref, out_ref,
              acc_ref, sem):
    fc = pl.program_id(0)
    t = pl.program_id(1)
    x = xs_ref[...]
    h = jnp.dot(x, w1_ref[0], preferred_element_type=jnp.float32)
    h = h / (1.0 + jnp.exp(-h))
    h = h * jnp.dot(x, w3_ref[0], preferred_element_type=jnp.float32)
    part = jnp.dot(h.astype(jnp.bfloat16), w2_ref[0],
                   preferred_element_type=jnp.float32)

    @pl.when(fc == 0)
    def _():
        acc_ref[t] = part

    @pl.when(fc > 0)
    def _():
        acc_ref[t] += part

    @pl.when(fc == NFC - 1)
    def _():
        acc_ref[t] *= wrow_ref[...]
        desc = pltpu.make_async_copy(
            acc_ref.at[t], out_ref.at[pl.ds(t * TM2, TM2)], sem)
        desc.start()
        desc.wait()


def _add_body(a_ref, b_ref, o_ref):
    o_ref[...] = a_ref[...] + b_ref[...]


def _run_router(x, Wg):
    return pl.pallas_call(
        _router_meta_body,
        grid=(NRB + 2 * NRB,),
        in_specs=[
            pl.BlockSpec((RB, D), lambda g: (jnp.minimum(g, NRB - 1), 0)),
            pl.BlockSpec((D, E), lambda g: (0, 0)),
        ],
        out_specs=[
            pl.BlockSpec((RB, E), lambda g: (jnp.minimum(g, NRB - 1), 0)),
            pl.BlockSpec((RB, 1), lambda g: (jnp.where(g < NRB, 0, g - NRB), 0)),
            pl.BlockSpec((RB, 1), lambda g: (jnp.where(g < NRB, 0, g - NRB), 0)),
            pl.BlockSpec((1, 128), lambda g: (0, 0)),
            pl.BlockSpec((RB, D), lambda g: (jnp.minimum(g, NRB - 1), 0)),
        ],
        out_shape=[
            jax.ShapeDtypeStruct((T, E), jnp.float32),
            jax.ShapeDtypeStruct((PAIRS, 1), jnp.int32),
            jax.ShapeDtypeStruct((PAIRS, 1), jnp.float32),
            jax.ShapeDtypeStruct((1, 128), jnp.int32),
            jax.ShapeDtypeStruct((T, D), jnp.bfloat16),
        ],
        scratch_shapes=[
            pltpu.VMEM((1, E), jnp.float32),
            pltpu.VMEM((1, E), jnp.float32),
            pltpu.VMEM((T, 1), jnp.int32),
            pltpu.VMEM((T, 1), jnp.int32),
            pltpu.VMEM((T, 1), jnp.float32),
        ],
    )(x, Wg)


def _run_gmm(gid, xs, W1, W3, W2, wrow):
    return pl.pallas_call(
        _gmm_body,
        grid_spec=pltpu.PrefetchScalarGridSpec(
            num_scalar_prefetch=1,
            grid=(NFC, NT),
            in_specs=[
                pl.BlockSpec((TM2, D), lambda fc, t, gid_r: (t, 0)),
                pl.BlockSpec((1, D, FC), lambda fc, t, gid_r: (gid_r[t], 0, fc)),
                pl.BlockSpec((1, D, FC), lambda fc, t, gid_r: (gid_r[t], 0, fc)),
                pl.BlockSpec((1, FC, D), lambda fc, t, gid_r: (gid_r[t], fc, 0)),
                pl.BlockSpec((TM2, 1), lambda fc, t, gid_r: (t, 0)),
            ],
            out_specs=pl.BlockSpec(memory_space=pl.ANY),
            scratch_shapes=[
                pltpu.VMEM((NT, TM2, D), jnp.float32),
                pltpu.SemaphoreType.DMA,
            ],
        ),
        out_shape=jax.ShapeDtypeStruct((R, D), jnp.float32),
        compiler_params=pltpu.CompilerParams(
            vmem_limit_bytes=100 * 1024 * 1024),
    )(gid, xs, W1, W3, W2, wrow.reshape(R, 1))


@jax.jit
def kernel(hidden_states, Wg, W1, W2, W3):
    b, s, d = hidden_states.shape
    x = hidden_states.reshape(-1, d)

    logits, pos2d, wp2d, gid2d, xb = _run_router(x, Wg)
    W1b = W1.astype(jnp.bfloat16)
    W3b = W3.astype(jnp.bfloat16)
    W2b = W2.astype(jnp.bfloat16)

    pos = pos2d.reshape(PAIRS)
    wp = wp2d.reshape(PAIRS)
    gid = gid2d.reshape(128)
    tok_pair = jnp.tile(jnp.arange(T, dtype=jnp.int32), K)
    zi = jnp.zeros((R,), jnp.int32)
    zf = jnp.zeros((R,), jnp.float32)

    mesh = plsc.VectorSubcoreMesh(core_axis_name="c", subcore_axis_name="s",
                                  num_cores=2, num_subcores=16)

    tokrow, wrow = pl.kernel(
        _scatter_body,
        out_type=[
            jax.ShapeDtypeStruct((R,), jnp.int32),
            jax.ShapeDtypeStruct((R,), jnp.float32),
        ],
        mesh=mesh,
        scratch_types=[
            pltpu.VMEM((PAIRS,), jnp.int32),
            pltpu.VMEM((PAIRS,), jnp.float32),
            pltpu.VMEM((PAIRS,), jnp.int32),
            pltpu.VMEM((R,), jnp.int32),
            pltpu.VMEM((R,), jnp.float32),
        ],
        compiler_params=pltpu.CompilerParams(needs_layout_passes=False),
    )(pos, wp, tok_pair, zi, zf)

    xb32 = lax.bitcast_convert_type(xb.reshape(T, D // 2, 2), jnp.int32)
    xs32 = pl.kernel(
        functools.partial(_gather_body, CW),
        out_type=jax.ShapeDtypeStruct((R, D // 2), jnp.int32),
        mesh=mesh,
        scratch_types=[
            pltpu.VMEM((CW,), jnp.int32),
            pltpu.VMEM((NBUF, 32, D // 2), jnp.int32),
        ] + [pltpu.SemaphoreType.DMA] * (2 * NBUF),
    )(xb32, tokrow)
    xs = lax.bitcast_convert_type(xs32, jnp.bfloat16).reshape(R, D)

    ysw = _run_gmm(gid, xs, W1b, W3b, W2b, wrow)

    yg = pl.kernel(
        functools.partial(_gather_body, CP),
        out_type=jax.ShapeDtypeStruct((PAIRS, D), jnp.float32),
        mesh=mesh,
        scratch_types=[
            pltpu.VMEM((CP,), jnp.int32),
            pltpu.VMEM((NBUF, 32, D), jnp.float32),
        ] + [pltpu.SemaphoreType.DMA] * (2 * NBUF),
    )(ysw, pos)

    final = pl.pallas_call(
        _add_body,
        grid=(T // TMA,),
        in_specs=[
            pl.BlockSpec((TMA, D), lambda i: (i, 0)),
            pl.BlockSpec((TMA, D), lambda i: (i + T // TMA, 0)),
        ],
        out_specs=pl.BlockSpec((TMA, D), lambda i: (i, 0)),
        out_shape=jax.ShapeDtypeStruct((T, D), jnp.float32),
    )(yg, yg)

    return final.reshape(b, s, d), logits


# speedup vs baseline: 1.5254x; 1.5254x over previous
import functools

import jax
import jax.numpy as jnp
from jax import lax
from jax.experimental import pallas as pl
from jax.experimental.pallas import tpu as pltpu
from jax.experimental.pallas import tpu_sc as plsc

T, D, FF, E, K = 4096, 1024, 4096, 8, 2
PAIRS = T * K
RB = 256
NRB = T // RB
TM2 = 128
R = PAIRS + E * TM2
NT = R // TM2
FC = 1024
NFC = FF // FC
NW = 32
CW = R // NW
CP = PAIRS // NW
TMA = 512


def _router_meta_body(x_ref, wg_ref, logits_ref, pos0_ref, wp0_ref, pos1_ref,
                      wp1_ref, gid_ref, cnt_ref, nxt_ref, idx1_ref, idx2_ref,
                      wa_ref):
    g = pl.program_id(0)
    eio = lax.broadcasted_iota(jnp.int32, (RB, E), 1)

    @pl.when(g == 0)
    def _():
        cnt_ref[...] = jnp.zeros_like(cnt_ref)

    @pl.when(g < NRB)
    def _():
        x = x_ref[...]
        logits = jnp.dot(x, wg_ref[...], preferred_element_type=jnp.float32)
        logits_ref[...] = logits
        m1 = jnp.max(logits, axis=1, keepdims=True)
        idx1 = jnp.min(jnp.where(logits == m1, eio, E), axis=1, keepdims=True)
        masked = jnp.where(eio == idx1, -1e30, logits)
        m2 = jnp.max(masked, axis=1, keepdims=True)
        idx2 = jnp.min(jnp.where(masked == m2, eio, E), axis=1, keepdims=True)
        wa = 1.0 / (1.0 + jnp.exp(m2 - m1))
        sl = pl.ds(g * RB, RB)
        idx1_ref[sl] = idx1
        idx2_ref[sl] = idx2
        wa_ref[sl] = wa
        oh = (eio == idx1).astype(jnp.float32) + (eio == idx2).astype(jnp.float32)
        cnt_ref[...] += jnp.sum(oh, axis=0, keepdims=True)
        pos0_ref[...] = jnp.zeros((RB, 1), jnp.int32)
        wp0_ref[...] = jnp.zeros((RB, 1), jnp.float32)
        pos1_ref[...] = jnp.zeros((RB, 1), jnp.int32)
        wp1_ref[...] = jnp.zeros((RB, 1), jnp.float32)

    @pl.when(g == NRB)
    def _():
        cnt = cnt_ref[...]
        pads = jnp.floor((cnt + (TM2 - 1)) * (1.0 / TM2)) * TM2
        rio8 = lax.broadcasted_iota(jnp.int32, (E, E), 0)
        cio8 = lax.broadcasted_iota(jnp.int32, (E, E), 1)
        mlt = (rio8 < cio8).astype(jnp.float32)
        excl = jnp.dot(pads, mlt, preferred_element_type=jnp.float32)
        nxt_ref[...] = excl
        tio = lax.broadcasted_iota(jnp.int32, (1, 128), 1).astype(jnp.float32) * TM2
        gid = jnp.zeros((1, 128), jnp.float32)
        for e in range(1, E):
            gid += (tio >= excl[0:1, e:e + 1]).astype(jnp.float32)
        gid_ref[...] = gid.astype(jnp.int32)

    @pl.when(g >= NRB)
    def _():
        j = g - NRB
        sl = pl.ds(j * RB, RB)
        idx1 = idx1_ref[sl]
        idx2 = idx2_ref[sl]
        wa = wa_ref[sl]
        rio = lax.broadcasted_iota(jnp.int32, (RB, RB), 0)
        cio = lax.broadcasted_iota(jnp.int32, (RB, RB), 1)
        lstrict = (cio < rio).astype(jnp.float32)
        nxt = nxt_ref[...]
        oh0 = (eio == idx1).astype(jnp.float32)
        csum0 = jnp.dot(lstrict, oh0, preferred_element_type=jnp.float32)
        rank0 = jnp.sum(csum0 * oh0, axis=1, keepdims=True)
        base0 = jnp.sum(nxt * oh0, axis=1, keepdims=True)
        pos0_ref[...] = (rank0 + base0).astype(jnp.int32)
        wp0_ref[...] = wa
        nxt = nxt + jnp.sum(oh0, axis=0, keepdims=True)
        oh1 = (eio == idx2).astype(jnp.float32)
        csum1 = jnp.dot(lstrict, oh1, preferred_element_type=jnp.float32)
        rank1 = jnp.sum(csum1 * oh1, axis=1, keepdims=True)
        base1 = jnp.sum(nxt * oh1, axis=1, keepdims=True)
        pos1_ref[...] = (rank1 + base1).astype(jnp.int32)
        wp1_ref[...] = 1.0 - wa
        nxt_ref[...] = nxt + jnp.sum(oh1, axis=0, keepdims=True)


def _scatter_body(pos_hbm, wp_hbm, tok_hbm, zi_hbm, zf_hbm,
                  tokrow_hbm, wrow_hbm,
                  pos_v, wp_v, tok_v, dtok_v, dw_v):
    c = lax.axis_index("c")
    s = lax.axis_index("s")
    wid = s * 2 + c

    @pl.when(wid == 0)
    def _():
        pltpu.sync_copy(pos_hbm, pos_v)
        pltpu.sync_copy(wp_hbm, wp_v)
        pltpu.sync_copy(tok_hbm, tok_v)
        pltpu.sync_copy(zi_hbm, dtok_v)
        pltpu.sync_copy(zf_hbm, dw_v)

        def sbody(i, carry):
            sl = pl.ds(i * 16, 16)
            idx = pos_v[sl]
            plsc.store_scatter(dtok_v, [idx], tok_v[sl])
            plsc.store_scatter(dw_v, [idx], wp_v[sl])
            return carry

        lax.fori_loop(0, PAIRS // 16, sbody, 0)
        pltpu.sync_copy(dtok_v, tokrow_hbm)
        pltpu.sync_copy(dw_v, wrow_hbm)


NBUF = 3


def _gather_body(rows_w, x_hbm, tok_hbm, xs_hbm, idx_v, buf_v,
                 semg0, semg1, semg2, semo0, semo1, semo2):
    c = lax.axis_index("c")
    s = lax.axis_index("s")
    wid = s * 2 + c
    base = wid * rows_w
    pltpu.sync_copy(tok_hbm.at[pl.ds(base, rows_w)], idx_v)
    semg = (semg0, semg1, semg2)
    semo = (semo0, semo1, semo2)
    nch = rows_w // 32
    gds = [None] * NBUF
    ods = [None] * NBUF
    for ch in range(nch):
        b = ch % NBUF
        if ods[b] is not None:
            ods[b].wait()
        gds[b] = pltpu.async_copy(
            x_hbm.at[idx_v.at[pl.ds(ch * 32, 32)]], buf_v.at[b], semg[b])
        old = ch - (NBUF - 1)
        if old >= 0:
            ob = old % NBUF
            gds[ob].wait()
            ods[ob] = pltpu.async_copy(
                buf_v.at[ob], xs_hbm.at[pl.ds(base + old * 32, 32)], semo[ob])
    for old in range(max(0, nch - (NBUF - 1)), nch):
        ob = old % NBUF
        gds[ob].wait()
        ods[ob] = pltpu.async_copy(
            buf_v.at[ob], xs_hbm.at[pl.ds(base + old * 32, 32)], semo[ob])
    for b in range(NBUF):
        if ods[b] is not None:
            ods[b].wait()


def _gmm_body(gid_ref, xs_ref, w1_ref, w3_ref, w2_ref, wrow_ref, out_ref,
              acc_ref, sem):
    fc = pl.program_id(0)
    t = pl.program_id(1)
    x = xs_ref[...]
    h = jnp.dot(x, w1_ref[0], preferred_element_type=jnp.float32)
    h = h / (1.0 + jnp.exp(-h))
    h = h * jnp.dot(x, w3_ref[0], preferred_element_type=jnp.float32)
    part = jnp.dot(h, w2_ref[0], preferred_element_type=jnp.float32)

    @pl.when(fc == 0)
    def _():
        acc_ref[t] = part

    @pl.when(fc > 0)
    def _():
        acc_ref[t] += part

    @pl.when(fc == NFC - 1)
    def _():
        acc_ref[t] *= wrow_ref[...]
        desc = pltpu.make_async_copy(
            acc_ref.at[t], out_ref.at[pl.ds(t * TM2, TM2)], sem)
        desc.start()
        desc.wait()


def _add_body(a_ref, b_ref, o_ref):
    o_ref[...] = a_ref[...] + b_ref[...]


def _run_router(x, Wg):
    pw_spec = pl.BlockSpec((RB, 1), lambda g: (jnp.where(g < NRB, 0, g - NRB), 0))
    logits, pos0, wp0, pos1, wp1, gid = pl.pallas_call(
        _router_meta_body,
        grid=(2 * NRB,),
        in_specs=[
            pl.BlockSpec((RB, D), lambda g: (jnp.minimum(g, NRB - 1), 0)),
            pl.BlockSpec((D, E), lambda g: (0, 0)),
        ],
        out_specs=[
            pl.BlockSpec((RB, E), lambda g: (jnp.minimum(g, NRB - 1), 0)),
            pw_spec, pw_spec, pw_spec, pw_spec,
            pl.BlockSpec((1, 128), lambda g: (0, 0)),
        ],
        out_shape=[
            jax.ShapeDtypeStruct((T, E), jnp.float32),
            jax.ShapeDtypeStruct((T, 1), jnp.int32),
            jax.ShapeDtypeStruct((T, 1), jnp.float32),
            jax.ShapeDtypeStruct((T, 1), jnp.int32),
            jax.ShapeDtypeStruct((T, 1), jnp.float32),
            jax.ShapeDtypeStruct((1, 128), jnp.int32),
        ],
        scratch_shapes=[
            pltpu.VMEM((1, E), jnp.float32),
            pltpu.VMEM((1, E), jnp.float32),
            pltpu.VMEM((T, 1), jnp.int32),
            pltpu.VMEM((T, 1), jnp.int32),
            pltpu.VMEM((T, 1), jnp.float32),
        ],
    )(x, Wg)
    pos = jnp.concatenate([pos0, pos1], axis=0)
    wp = jnp.concatenate([wp0, wp1], axis=0)
    return logits, pos, wp, gid


def _run_gmm(gid, xs, W1, W3, W2, wrow):
    return pl.pallas_call(
        _gmm_body,
        grid_spec=pltpu.PrefetchScalarGridSpec(
            num_scalar_prefetch=1,
            grid=(NFC, NT),
            in_specs=[
                pl.BlockSpec((TM2, D), lambda fc, t, gid_r: (t, 0)),
                pl.BlockSpec((1, D, FC), lambda fc, t, gid_r: (gid_r[t], 0, fc)),
                pl.BlockSpec((1, D, FC), lambda fc, t, gid_r: (gid_r[t], 0, fc)),
                pl.BlockSpec((1, FC, D), lambda fc, t, gid_r: (gid_r[t], fc, 0)),
                pl.BlockSpec((TM2, 1), lambda fc, t, gid_r: (t, 0)),
            ],
            out_specs=pl.BlockSpec(memory_space=pl.ANY),
            scratch_shapes=[
                pltpu.VMEM((NT, TM2, D), jnp.float32),
                pltpu.SemaphoreType.DMA,
            ],
        ),
        out_shape=jax.ShapeDtypeStruct((R, D), jnp.float32),
        compiler_params=pltpu.CompilerParams(
            vmem_limit_bytes=100 * 1024 * 1024),
    )(gid, xs, W1, W3, W2, wrow.reshape(R, 1))


@jax.jit
def kernel(hidden_states, Wg, W1, W2, W3):
    b, s, d = hidden_states.shape
    x = hidden_states.reshape(-1, d)

    logits, pos2d, wp2d, gid2d = _run_router(x, Wg)

    pos = pos2d.reshape(PAIRS)
    wp = wp2d.reshape(PAIRS)
    gid = gid2d.reshape(128)
    tok_pair = jnp.tile(jnp.arange(T, dtype=jnp.int32), K)
    zi = jnp.zeros((R,), jnp.int32)
    zf = jnp.zeros((R,), jnp.float32)

    mesh = plsc.VectorSubcoreMesh(core_axis_name="c", subcore_axis_name="s",
                                  num_cores=2, num_subcores=16)

    tokrow, wrow = pl.kernel(
        _scatter_body,
        out_type=[
            jax.ShapeDtypeStruct((R,), jnp.int32),
            jax.ShapeDtypeStruct((R,), jnp.float32),
        ],
        mesh=mesh,
        scratch_types=[
            pltpu.VMEM((PAIRS,), jnp.int32),
            pltpu.VMEM((PAIRS,), jnp.float32),
            pltpu.VMEM((PAIRS,), jnp.int32),
            pltpu.VMEM((R,), jnp.int32),
            pltpu.VMEM((R,), jnp.float32),
        ],
        compiler_params=pltpu.CompilerParams(needs_layout_passes=False),
    )(pos, wp, tok_pair, zi, zf)

    xs = pl.kernel(
        functools.partial(_gather_body, CW),
        out_type=jax.ShapeDtypeStruct((R, D), jnp.float32),
        mesh=mesh,
        scratch_types=[
            pltpu.VMEM((CW,), jnp.int32),
            pltpu.VMEM((NBUF, 32, D), jnp.float32),
        ] + [pltpu.SemaphoreType.DMA] * (2 * NBUF),
    )(x, tokrow)

    ysw = _run_gmm(gid, xs, W1, W3, W2, wrow)

    yg = pl.kernel(
        functools.partial(_gather_body, CP),
        out_type=jax.ShapeDtypeStruct((PAIRS, D), jnp.float32),
        mesh=mesh,
        scratch_types=[
            pltpu.VMEM((CP,), jnp.int32),
            pltpu.VMEM((NBUF, 32, D), jnp.float32),
        ] + [pltpu.SemaphoreType.DMA] * (2 * NBUF),
    )(ysw, pos)

    final = pl.pallas_call(
        _add_body,
        grid=(T // TMA,),
        in_specs=[
            pl.BlockSpec((TMA, D), lambda i: (i, 0)),
            pl.BlockSpec((TMA, D), lambda i: (i + T // TMA, 0)),
        ],
        out_specs=pl.BlockSpec((TMA, D), lambda i: (i, 0)),
        out_shape=jax.ShapeDtypeStruct((T, D), jnp.float32),
    )(yg, yg)

    return final.reshape(b, s, d), logits
